# baseline (speedup 1.0000x reference)
import functools
import jax
import jax.numpy as jnp
from jax import lax
from jax.experimental import pallas as pl
from jax.experimental.pallas import tpu as pltpu
from jax.experimental.pallas import tpu_sc as plsc

_N = 32768
_D = 1024
_NC = 2
_NS = 16
_NW = _NC * _NS
_RPW = _N // _NW
_CHUNK = 16
_NCHUNK = _RPW // _CHUNK
_VPR = _D // 16


def _sc_body(x_hbm, p0_hbm, p1_hbm, pe_hbm, out_hbm,
             posb, idxa, idxb, xbuf, ga, gb, sem_x, sem_a, sem_b, sem_o):
    cid = lax.axis_index("c")
    sid = lax.axis_index("s")
    wid = sid * _NC + cid
    base = wid * _RPW

    pltpu.sync_copy(p0_hbm.at[pl.ds(base, _RPW)], posb)

    def cvt_a(i, _):
        v = posb[pl.ds(i * 16, 16)]
        idxa[pl.ds(i * 16, 16)] = jnp.clip((v * 100.0).astype(jnp.int32), 0, 199)
        return 0

    lax.fori_loop(0, _RPW // 16, cvt_a, 0, unroll=8)

    pltpu.sync_copy(p1_hbm.at[pl.ds(base, _RPW)], posb)

    def cvt_b(i, _):
        v = posb[pl.ds(i * 16, 16)]
        idxb[pl.ds(i * 16, 16)] = jnp.clip((v * 100.0).astype(jnp.int32), 0, 199)
        return 0

    lax.fori_loop(0, _RPW // 16, cvt_b, 0, unroll=8)

    def chunk_body(c, _):
        off = c * _CHUNK
        cpx = pltpu.async_copy(x_hbm.at[pl.ds(base + off, _CHUNK)], xbuf, sem_x)
        cpa = pltpu.async_copy(pe_hbm.at[idxa.at[pl.ds(off, _CHUNK)]], ga, sem_a)
        cpb = pltpu.async_copy(pe_hbm.at[idxb.at[pl.ds(off, _CHUNK)]], gb, sem_b)
        cpx.wait()
        cpa.wait()
        cpb.wait()

        def row_body(r, _):
            def vec_body(i, _):
                o = i * 16
                ga[r, pl.ds(o, 16)] = (
                    ga[r, pl.ds(o, 16)] + gb[r, pl.ds(o, 16)] + xbuf[r, pl.ds(o, 16)]
                )
                return 0

            lax.fori_loop(0, _VPR, vec_body, 0, unroll=8)
            return 0

        lax.fori_loop(0, _CHUNK, row_body, 0)
        pltpu.async_copy(ga, out_hbm.at[pl.ds(base + off, _CHUNK)], sem_o).wait()
        return 0

    lax.fori_loop(0, _NCHUNK, chunk_body, 0)


def kernel(x, positions, pe):
    b, s, d = x.shape
    n = b * s
    x2 = x.reshape(n, d)
    p0 = positions[..., 0].reshape(n)
    p1 = positions[..., 1].reshape(n)

    mesh = plsc.VectorSubcoreMesh(core_axis_name="c", subcore_axis_name="s")
    fn = functools.partial(
        pl.kernel,
        mesh=mesh,
        out_type=jax.ShapeDtypeStruct((n, d), x.dtype),
        scratch_types=[
            pltpu.VMEM((_RPW,), jnp.float32),
            pltpu.VMEM((_RPW,), jnp.int32),
            pltpu.VMEM((_RPW,), jnp.int32),
            pltpu.VMEM((_CHUNK, _D), jnp.float32),
            pltpu.VMEM((_CHUNK, _D), jnp.float32),
            pltpu.VMEM((_CHUNK, _D), jnp.float32),
            pltpu.SemaphoreType.DMA,
            pltpu.SemaphoreType.DMA,
            pltpu.SemaphoreType.DMA,
            pltpu.SemaphoreType.DMA,
        ],
    )(_sc_body)
    out = fn(x2, p0, p1, pe)
    return out.reshape(b, s, d)
